# baseline (speedup 1.0000x reference)
import functools

import jax
import jax.numpy as jnp
from jax import lax
from jax.experimental import pallas as pl
from jax.experimental.pallas import tpu as pltpu
from jax.experimental.pallas import tpu_sc as plsc

D = 64
DP = 65
NW = 32
CHUNK = 512
L = 16


@functools.partial(jax.jit, static_argnums=(2,))
def _sc_lookup(table_pad, idx_flat, n):
    bpw = n // NW
    nchunk = bpw // CHUNK
    tw = table_pad.shape[0]
    mesh = plsc.VectorSubcoreMesh(core_axis_name="c", subcore_axis_name="s")

    @functools.partial(
        pl.kernel,
        out_type=jax.ShapeDtypeStruct((n // 2, 2 * D), jnp.float32),
        mesh=mesh,
        scratch_types=[
            pltpu.VMEM((tw,), jnp.float32),
            pltpu.VMEM((bpw // 128, 128), jnp.int32),
            pltpu.VMEM((2, CHUNK // 2, 2 * DP), jnp.float32),
            pltpu.SemaphoreType.DMA,
        ],
        compiler_params=pltpu.CompilerParams(
            use_tc_tiling_on_sc=False, needs_layout_passes=False
        ),
    )
    def k(table_hbm, idx_hbm, out_hbm, table_v, idx_v, buf_v, wsem):
        wid = lax.axis_index("s") * 2 + lax.axis_index("c")
        base = wid * bpw
        pltpu.sync_copy(table_hbm, table_v)
        pltpu.sync_copy(idx_hbm.at[pl.ds(wid * (bpw // 128), bpw // 128)], idx_v)

        lanes = lax.iota(jnp.int32, L)
        lane_row2 = lax.shift_right_logical(lanes, 1)
        lane_colb = (lanes & 1) * DP

        def write(c, b, start):
            cp = pltpu.make_async_copy(
                buf_v.at[b, :, pl.ds(0, 2 * D)],
                out_hbm.at[pl.ds((base + c * CHUNK) // 2, CHUNK // 2)],
                wsem,
            )
            cp.start() if start else cp.wait()

        def compute(c, b):
            bufb = buf_v.at[b]

            def group(g, carry):
                row = c * (CHUNK // 128) + lax.shift_right_logical(g, 3)
                idx16 = idx_v[row, pl.ds((g & 7) * L, L)]
                src0 = idx16 * DP
                row2 = g * (L // 2) + lane_row2
                for blk in range(0, D, 16):
                    vals = [
                        plsc.load_gather(table_v, [src0 + col])
                        for col in range(blk, blk + 16)
                    ]
                    for i, col in enumerate(range(blk, blk + 16)):
                        plsc.store_scatter(bufb, [row2, lane_colb + col], vals[i])
                return carry

            lax.fori_loop(0, CHUNK // L, group, 0)

        def body(o, carry):
            for b in range(2):
                c = o * 2 + b
                compute(c, b)
                pl.when(c >= 1)(lambda: write(c - 1, (b + 1) % 2, False))
                write(c, b, True)
            return carry

        lax.fori_loop(0, nchunk // 2, body, 0)
        write(nchunk - 1, (nchunk - 1) % 2, False)

    return k(table_pad, idx_flat)


def kernel(char_idx, table):
    b, s, t = char_idx.shape
    n = b * s * t
    idx_2d = char_idx.reshape(n // 128, 128).astype(jnp.int32)
    table_pad = jnp.pad(table, ((0, 0), (0, DP - D))).reshape(-1)
    out = _sc_lookup(table_pad, idx_2d, n)
    return out.reshape(b, s, t, D)


# speedup vs baseline: 1.0056x; 1.0056x over previous
import functools

import jax
import jax.numpy as jnp
from jax import lax
from jax.experimental import pallas as pl
from jax.experimental.pallas import tpu as pltpu
from jax.experimental.pallas import tpu_sc as plsc

D = 64
DP = 65
NW = 32
CHUNK = 512
L = 16


@functools.partial(jax.jit, static_argnums=(2,))
def _sc_lookup(table_pad, idx_flat, n):
    bpw = n // NW
    nchunk = bpw // CHUNK
    tw = table_pad.shape[0]
    mesh = plsc.VectorSubcoreMesh(core_axis_name="c", subcore_axis_name="s")

    @functools.partial(
        pl.kernel,
        out_type=jax.ShapeDtypeStruct((n // 2, 2 * D), jnp.float32),
        mesh=mesh,
        scratch_types=[
            pltpu.VMEM((tw,), jnp.float32),
            pltpu.VMEM((bpw,), jnp.int32),
            pltpu.VMEM((2, CHUNK // 2, 2 * DP), jnp.float32),
            pltpu.SemaphoreType.DMA,
        ],
        compiler_params=pltpu.CompilerParams(
            use_tc_tiling_on_sc=False, needs_layout_passes=False
        ),
    )
    def k(table_hbm, idx_hbm, out_hbm, table_v, idx_v, buf_v, wsem):
        wid = lax.axis_index("s") * 2 + lax.axis_index("c")
        base = wid * bpw
        pltpu.sync_copy(table_hbm, table_v)
        pltpu.sync_copy(idx_hbm.at[pl.ds(base, bpw)], idx_v)

        lanes = lax.iota(jnp.int32, L)
        lane_row2 = lax.shift_right_logical(lanes, 1)
        lane_colb = (lanes & 1) * DP

        def write(c, b, start):
            cp = pltpu.make_async_copy(
                buf_v.at[b, :, pl.ds(0, 2 * D)],
                out_hbm.at[pl.ds((base + c * CHUNK) // 2, CHUNK // 2)],
                wsem,
            )
            cp.start() if start else cp.wait()

        def compute(c, b):
            bufb = buf_v.at[b]

            def group(g, carry):
                src0 = idx_v[pl.ds(c * CHUNK + g * L, L)]
                row2 = g * (L // 2) + lane_row2
                for blk in range(0, D, 16):
                    vals = [
                        plsc.load_gather(table_v, [src0 + col])
                        for col in range(blk, blk + 16)
                    ]
                    for i, col in enumerate(range(blk, blk + 16)):
                        plsc.store_scatter(bufb, [row2, lane_colb + col], vals[i])
                return carry

            lax.fori_loop(0, CHUNK // L, group, 0)

        def body(o, carry):
            for b in range(2):
                c = o * 2 + b
                compute(c, b)
                pl.when(c >= 1)(lambda: write(c - 1, (b + 1) % 2, False))
                write(c, b, True)
            return carry

        lax.fori_loop(0, nchunk // 2, body, 0)
        write(nchunk - 1, (nchunk - 1) % 2, False)

    return k(table_pad, idx_flat)


def kernel(char_idx, table):
    b, s, t = char_idx.shape
    n = b * s * t
    idx_flat = char_idx.reshape(-1).astype(jnp.int32) * DP
    table_pad = jnp.pad(table, ((0, 0), (0, DP - D))).reshape(-1)
    out = _sc_lookup(table_pad, idx_flat, n)
    return out.reshape(b, s, t, D)


# speedup vs baseline: 1.0688x; 1.0629x over previous
import functools

import jax
import jax.numpy as jnp
from jax import lax
from jax.experimental import pallas as pl
from jax.experimental.pallas import tpu as pltpu
from jax.experimental.pallas import tpu_sc as plsc

D = 64
NW = 32
CHUNK = 512
L = 16
TPAD = 17040


@functools.partial(jax.jit, static_argnums=(2,))
def _sc_lookup(table_pad, idx_flat, n):
    bpw = n // NW
    nchunk = bpw // CHUNK
    mesh = plsc.VectorSubcoreMesh(core_axis_name="c", subcore_axis_name="s")

    @functools.partial(
        pl.kernel,
        out_type=jax.ShapeDtypeStruct((n * D,), jnp.float32),
        mesh=mesh,
        scratch_types=[
            pltpu.VMEM((TPAD,), jnp.float32),
            pltpu.VMEM((bpw,), jnp.int32),
            pltpu.VMEM((2, CHUNK * D), jnp.float32),
            pltpu.SemaphoreType.DMA,
        ],
        compiler_params=pltpu.CompilerParams(
            use_tc_tiling_on_sc=False, needs_layout_passes=False
        ),
    )
    def k(table_hbm, idx_hbm, out_hbm, table_v, idx_v, buf_v, wsem):
        wid = lax.axis_index("s") * 2 + lax.axis_index("c")
        base = wid * bpw
        pltpu.sync_copy(table_hbm, table_v)
        pltpu.sync_copy(idx_hbm.at[pl.ds(base, bpw)], idx_v)

        lanes = lax.iota(jnp.int32, L)

        def write(c, b, start):
            cp = pltpu.make_async_copy(
                buf_v.at[b],
                out_hbm.at[pl.ds((base + c * CHUNK) * D, CHUNK * D)],
                wsem,
            )
            cp.start() if start else cp.wait()

        def compute(c, b):
            bufb = buf_v.at[b]

            def group(g, carry):
                idx16 = idx_v[pl.ds(c * CHUNK + g * L, L)]
                src0 = idx16 * 65
                dst0 = (g * L + lanes) * D
                for k0 in range(0, D, 16):
                    diags = [(lanes + (k0 + j)) & 63 for j in range(16)]
                    vals = [
                        plsc.load_gather(table_v, [src0 + dg]) for dg in diags
                    ]
                    for dg, v in zip(diags, vals):
                        plsc.store_scatter(bufb, [dst0 + dg], v)
                return carry

            lax.fori_loop(0, CHUNK // L, group, 0)

        def body(o, carry):
            for b in range(2):
                c = o * 2 + b
                compute(c, b)
                pl.when(c >= 1)(lambda: write(c - 1, (b + 1) % 2, False))
                write(c, b, True)
            return carry

        lax.fori_loop(0, nchunk // 2, body, 0)
        write(nchunk - 1, (nchunk - 1) % 2, False)

    return k(table_pad, idx_flat)


def kernel(char_idx, table):
    b, s, t = char_idx.shape
    n = b * s * t
    idx_flat = char_idx.reshape(-1).astype(jnp.int32)
    table_pad = jnp.pad(table, ((0, 0), (0, 1))).reshape(-1)
    table_pad = jnp.pad(table_pad, (0, TPAD - table_pad.shape[0]))
    out = _sc_lookup(table_pad, idx_flat, n)
    return out.reshape(b, s, t, D)
